# Initial kernel scaffold; baseline (speedup 1.0000x reference)
import functools

import jax
import jax.numpy as jnp
from jax import lax
from jax.experimental import pallas as pl
from jax.experimental.pallas import tpu as pltpu
from jax.experimental.pallas import tpu_sc as plsc

N = 4096
E = 131072
HID = 128
NH = 8
HD = 16

NC = 2
NS = 16
LANES = 16
NW = NC * NS
EPW = E // NW
CHUNK = 128
NCHUNK = EPW // CHUNK
RPT = N // NS
VPR = HID // LANES

F32 = jnp.float32


def _sc_mesh():
    return plsc.VectorSubcoreMesh(core_axis_name="c", subcore_axis_name="s")


def _prep_body(nf, wne, bne, wm1, wm2, h_out, hs_out, hd_out):
    h = jnp.dot(nf[...], wne[...], preferred_element_type=F32) + bne[...]
    h_out[...] = h
    hs_out[...] = jnp.dot(h, wm1[...], preferred_element_type=F32)
    hd_out[...] = jnp.dot(h, wm2[...], preferred_element_type=F32)


def _prep_call(nf, wne, bne, wm1, wm2):
    out = [jax.ShapeDtypeStruct((N, HID), F32)] * 3
    return pl.pallas_call(_prep_body, out_shape=out)(nf, wne, bne, wm1, wm2)


_EM_TILE = 4096


def _em_body(ef, wee, bee, wm3c, bmc, em0, em1, em2):
    wem = jnp.dot(wee[...], wm3c[...], preferred_element_type=F32)
    bem = jnp.dot(bee[...], wm3c[...], preferred_element_type=F32) + bmc[...]
    z = jnp.dot(ef[...], wem, preferred_element_type=F32) + bem
    em0[...] = z[:, :HID]
    em1[...] = z[:, HID:2 * HID]
    em2[...] = z[:, 2 * HID:]


def _em_call(ef, wee, bee, wm3c, bmc):
    grid = (E // _EM_TILE,)
    out = [jax.ShapeDtypeStruct((E, HID), F32)] * 3
    return pl.pallas_call(
        _em_body,
        grid=grid,
        in_specs=[
            pl.BlockSpec((_EM_TILE, 16), lambda i: (i, 0)),
            pl.BlockSpec((16, 3 * HID), lambda i: (0, 0)),
            pl.BlockSpec((1, 16), lambda i: (0, 0)),
            pl.BlockSpec((HID, 3 * HID), lambda i: (0, 0)),
            pl.BlockSpec((1, 3 * HID), lambda i: (0, 0)),
        ],
        out_specs=[pl.BlockSpec((_EM_TILE, HID), lambda i: (i, 0))] * 3,
        out_shape=out,
    )(ef, wee, bee, wm3c, bmc)


def _upd_first_body(h, accp, cntp, wu1, wu2, bu, wn1, wn2,
                    hn_out, hsn_out, hdn_out, rinv_out):
    cw = cntp[...]
    cnt = cw[0, :, 0:1] + cw[1, :, 0:1]
    rinv = 1.0 / (cnt + 1e-6)
    rinv_out[...] = rinv
    agg = (accp[0] + accp[1]) * rinv
    hn = jnp.maximum(
        jnp.dot(h[...], wu1[...], preferred_element_type=F32)
        + jnp.dot(agg, wu2[...], preferred_element_type=F32) + bu[...], 0.0)
    hn_out[...] = hn
    hsn_out[...] = jnp.dot(hn, wn1[...], preferred_element_type=F32)
    hdn_out[...] = jnp.dot(hn, wn2[...], preferred_element_type=F32)


def _upd_mid_body(h, accp, rinv, wu1, wu2, bu, wn1, wn2,
                  hn_out, hsn_out, hdn_out):
    agg = (accp[0] + accp[1]) * rinv[...]
    hn = jnp.maximum(
        jnp.dot(h[...], wu1[...], preferred_element_type=F32)
        + jnp.dot(agg, wu2[...], preferred_element_type=F32) + bu[...], 0.0)
    hn_out[...] = hn
    hsn_out[...] = jnp.dot(hn, wn1[...], preferred_element_type=F32)
    hdn_out[...] = jnp.dot(hn, wn2[...], preferred_element_type=F32)


def _upd_last_body(h, accp, rinv, wu1, wu2, bu, wqkv, bqkv, qkv_out):
    agg = (accp[0] + accp[1]) * rinv[...]
    hn = jnp.maximum(
        jnp.dot(h[...], wu1[...], preferred_element_type=F32)
        + jnp.dot(agg, wu2[...], preferred_element_type=F32) + bu[...], 0.0)
    qkv_out[...] = jnp.dot(hn, wqkv[...], preferred_element_type=F32) + bqkv[...]


_TQ = 256


def _attn_body(qkv_q, qkv_all, wo, bo, wh4, bh4, att_out, attw_out, heads_out):
    q = qkv_q[...]
    kv = qkv_all[...]
    accw = jnp.zeros((_TQ, N), F32)
    ctxs = []
    for hh in range(NH):
        qh = q[:, hh * HD:(hh + 1) * HD]
        kh = kv[:, HID + hh * HD: HID + (hh + 1) * HD]
        vh = kv[:, 2 * HID + hh * HD: 2 * HID + (hh + 1) * HD]
        s = lax.dot_general(qh, kh, (((1,), (1,)), ((), ())),
                            preferred_element_type=F32) * 0.25
        s = s - jnp.max(s, axis=1, keepdims=True)
        pexp = jnp.exp(s)
        attn = pexp / jnp.sum(pexp, axis=1, keepdims=True)
        accw = accw + attn
        ctxs.append(jnp.dot(attn, vh, preferred_element_type=F32))
    ctx = jnp.concatenate(ctxs, axis=1)
    att = jnp.dot(ctx, wo[...], preferred_element_type=F32) + bo[...]
    att_out[...] = att
    attw_out[...] = accw * (1.0 / NH)
    z = jnp.dot(att, wh4[...], preferred_element_type=F32) + bh4[...]
    sig = 1.0 / (1.0 + jnp.exp(-z))
    heads_out[...] = jnp.concatenate([sig[:, :2], z[:, 2:]], axis=1)


def _attn_call(qkv, wo, bo, wh4, bh4):
    grid = (N // _TQ,)
    out = [
        jax.ShapeDtypeStruct((N, HID), F32),
        jax.ShapeDtypeStruct((N, N), F32),
        jax.ShapeDtypeStruct((N, 4), F32),
    ]
    return pl.pallas_call(
        _attn_body,
        grid=grid,
        in_specs=[
            pl.BlockSpec((_TQ, 3 * HID), lambda i: (i, 0)),
            pl.BlockSpec((N, 3 * HID), lambda i: (0, 0)),
            pl.BlockSpec((HID, HID), lambda i: (0, 0)),
            pl.BlockSpec((1, HID), lambda i: (0, 0)),
            pl.BlockSpec((HID, 4), lambda i: (0, 0)),
            pl.BlockSpec((1, 4), lambda i: (0, 0)),
        ],
        out_specs=[
            pl.BlockSpec((_TQ, HID), lambda i: (i, 0)),
            pl.BlockSpec((_TQ, N), lambda i: (i, 0)),
            pl.BlockSpec((_TQ, 4), lambda i: (i, 0)),
        ],
        out_shape=out,
    )(qkv, wo, bo, wh4, bh4)


def _make_sc_layer(with_cnt):
    out_type = [jax.ShapeDtypeStruct((NC, N, HID), F32)]
    scratch = [
        pltpu.VMEM((CHUNK,), jnp.int32),
        pltpu.VMEM((CHUNK,), jnp.int32),
        pltpu.VMEM((CHUNK, HID), F32),
        pltpu.VMEM((CHUNK, HID), F32),
        pltpu.VMEM((CHUNK, HID), F32),
        pltpu.VMEM_SHARED((N, HID), F32),
        pltpu.SemaphoreType.DMA,
        pltpu.SemaphoreType.DMA,
        pltpu.SemaphoreType.DMA,
    ]
    if with_cnt:
        out_type.append(jax.ShapeDtypeStruct((NC, N, LANES), F32))
        scratch += [
            pltpu.VMEM((CHUNK, LANES), F32),
            pltpu.VMEM((RPT, LANES), F32),
            pltpu.VMEM_SHARED((N, LANES), F32),
        ]

    def body(row_h, col_h, hs_h, hd_h, em_h, *rest):
        if with_cnt:
            (acc_out, cnt_out, riv, civ, srcv, dstv, emv, accS,
             s1, s2, s3, onesv, zc16, cntS) = rest
        else:
            (acc_out, riv, civ, srcv, dstv, emv, accS, s1, s2, s3) = rest
        cid = lax.axis_index("c")
        sid = lax.axis_index("s")
        wid = cid * NS + sid
        base_r = sid * RPT

        zero = jnp.zeros((LANES,), F32)

        def zb(i, _):
            e = i // VPR
            j = lax.rem(i, VPR) * LANES
            srcv[e, pl.ds(j, LANES)] = zero
            return 0
        lax.fori_loop(0, CHUNK * VPR, zb, 0)
        pltpu.sync_copy(srcv, accS.at[pl.ds(base_r, CHUNK)])
        pltpu.sync_copy(srcv, accS.at[pl.ds(base_r + CHUNK, CHUNK)])
        if with_cnt:
            one = jnp.ones((LANES,), F32)

            def ob(i, _):
                onesv[i, :] = one
                return 0
            lax.fori_loop(0, CHUNK, ob, 0)

            def zcb(i, _):
                zc16[i, :] = zero
                return 0
            lax.fori_loop(0, RPT, zcb, 0)
            pltpu.sync_copy(zc16, cntS.at[pl.ds(base_r, RPT)])
        plsc.subcore_barrier()

        def chunk_body(t, _):
            base = wid * EPW + t * CHUNK
            pltpu.sync_copy(row_h.at[pl.ds(base, CHUNK)], riv)
            pltpu.sync_copy(col_h.at[pl.ds(base, CHUNK)], civ)
            c1 = pltpu.async_copy(hs_h.at[riv], srcv, s1)
            c2 = pltpu.async_copy(hd_h.at[civ], dstv, s2)
            c3 = pltpu.async_copy(em_h.at[pl.ds(base, CHUNK)], emv, s3)
            c1.wait()
            c2.wait()
            c3.wait()

            def eb(e, _):
                for j in range(VPR):
                    sl = pl.ds(j * LANES, LANES)
                    x = srcv[e, sl] + dstv[e, sl] + emv[e, sl]
                    emv[e, sl] = jnp.maximum(x, 0.0)
                return 0
            lax.fori_loop(0, CHUNK, eb, 0)
            pltpu.sync_copy(emv, accS.at[civ], add=True)
            if with_cnt:
                pltpu.sync_copy(onesv, cntS.at[civ], add=True)
            return 0
        lax.fori_loop(0, NCHUNK, chunk_body, 0)
        plsc.subcore_barrier()
        pltpu.sync_copy(accS.at[pl.ds(base_r, RPT)],
                        acc_out.at[cid, pl.ds(base_r, RPT)])
        if with_cnt:
            pltpu.sync_copy(cntS.at[pl.ds(base_r, RPT)],
                            cnt_out.at[cid, pl.ds(base_r, RPT)])

    return pl.kernel(body, out_type=tuple(out_type),
                     mesh=_sc_mesh(), scratch_types=scratch)


def _make_sc_atk():
    scratch = [
        pltpu.VMEM((N,), F32),
        pltpu.VMEM((N,), F32),
        pltpu.VMEM((EPW,), jnp.int32),
        pltpu.VMEM((EPW,), jnp.int32),
        pltpu.VMEM((EPW,), F32),
    ]

    def body(asrc_h, adst_h, row_h, col_h, out_h, av, bv, riv, civ, ov):
        cid = lax.axis_index("c")
        sid = lax.axis_index("s")
        wid = cid * NS + sid
        base = wid * EPW
        pltpu.sync_copy(asrc_h, av)
        pltpu.sync_copy(adst_h, bv)
        pltpu.sync_copy(row_h.at[pl.ds(base, EPW)], riv)
        pltpu.sync_copy(col_h.at[pl.ds(base, EPW)], civ)

        def eb(i, _):
            sl = pl.ds(i * LANES, LANES)
            a = plsc.load_gather(av, [riv[sl]])
            b = plsc.load_gather(bv, [civ[sl]])
            x = a + b
            ov[sl] = 1.0 / (1.0 + jnp.exp(-x))
            return 0
        lax.fori_loop(0, EPW // LANES, eb, 0)
        pltpu.sync_copy(ov, out_h.at[pl.ds(base, EPW)])

    return pl.kernel(body, out_type=jax.ShapeDtypeStruct((E,), F32),
                     mesh=_sc_mesh(), scratch_types=scratch)


def kernel(node_features, edge_features, edge_index, params):
    p = params
    layers = p['layers']
    row = edge_index[0]
    col = edge_index[1]

    wm1 = [lp['Wm'][:HID] for lp in layers]
    wm2 = [lp['Wm'][HID:2 * HID] for lp in layers]
    wm3c = jnp.concatenate([lp['Wm'][2 * HID:] for lp in layers], axis=1)
    bmc = jnp.concatenate([lp['bm'] for lp in layers])[None, :]
    wu1 = [lp['Wu'][:HID] for lp in layers]
    wu2 = [lp['Wu'][HID:] for lp in layers]
    bu = [lp['bu'][None, :] for lp in layers]
    wqkv = jnp.concatenate([p['Wq'], p['Wk'], p['Wv']], axis=1)
    bqkv = jnp.concatenate([p['bq'], p['bk'], p['bv']])[None, :]
    wh4 = jnp.concatenate(
        [p['Wvul'], p['Wdef'], p['Watk'][:HID], p['Watk'][HID:]], axis=1)
    bh4 = jnp.concatenate(
        [p['bvul'], p['bdef'], p['batk'], jnp.zeros((1,), F32)])[None, :]

    h, hs, hd = _prep_call(node_features, p['W_ne'], p['b_ne'][None, :],
                           wm1[0], wm2[0])
    ems = _em_call(edge_features, p['W_ee'], p['b_ee'][None, :], wm3c, bmc)

    sc_first = _make_sc_layer(True)
    sc_rest = _make_sc_layer(False)

    upd_first = pl.pallas_call(_upd_first_body, out_shape=[
        jax.ShapeDtypeStruct((N, HID), F32),
        jax.ShapeDtypeStruct((N, HID), F32),
        jax.ShapeDtypeStruct((N, HID), F32),
        jax.ShapeDtypeStruct((N, 1), F32),
    ])
    upd_mid = pl.pallas_call(_upd_mid_body, out_shape=[
        jax.ShapeDtypeStruct((N, HID), F32),
        jax.ShapeDtypeStruct((N, HID), F32),
        jax.ShapeDtypeStruct((N, HID), F32),
    ])
    upd_last = pl.pallas_call(_upd_last_body, out_shape=[
        jax.ShapeDtypeStruct((N, 3 * HID), F32),
    ])

    accp, cntp = sc_first(row, col, hs, hd, ems[0])
    h, hs, hd, rinv = upd_first(h, accp, cntp, wu1[0], wu2[0], bu[0],
                                wm1[1], wm2[1])

    accp = sc_rest(row, col, hs, hd, ems[1])[0]
    h, hs, hd = upd_mid(h, accp, rinv, wu1[1], wu2[1], bu[1],
                        wm1[2], wm2[2])

    accp = sc_rest(row, col, hs, hd, ems[2])[0]
    qkv = upd_last(h, accp, rinv, wu1[2], wu2[2], bu[2], wqkv, bqkv)[0]

    attended, attn_w, heads = _attn_call(qkv, p['Wo'], p['bo'][None, :],
                                         wh4, bh4)
    vuln = heads[:, 0:1]
    defp = heads[:, 1:2]
    asrc = heads[:, 2]
    adst = heads[:, 3]

    atk = _make_sc_atk()(asrc, adst, row, col)[:, None]
    return (attended, vuln, defp, atk, attn_w)


# speedup vs baseline: 5.2653x; 5.2653x over previous
import functools

import jax
import jax.numpy as jnp
from jax import lax
from jax.experimental import pallas as pl
from jax.experimental.pallas import tpu as pltpu
from jax.experimental.pallas import tpu_sc as plsc

N = 4096
E = 131072
HID = 128
NH = 8
HD = 16

NC = 2
NS = 16
LANES = 16
NW = NC * NS
EPW = E // NW
CHUNK = 128
NCHUNK = EPW // CHUNK
RPT = N // NS
VPR = HID // LANES

F32 = jnp.float32


def _sc_mesh():
    return plsc.VectorSubcoreMesh(core_axis_name="c", subcore_axis_name="s")


def _prep_body(nf, wne, bne, wm1, wm2, h_out, hs_out, hd_out):
    h = jnp.dot(nf[...], wne[...], preferred_element_type=F32) + bne[...]
    h_out[...] = h
    hs_out[...] = jnp.dot(h, wm1[...], preferred_element_type=F32)
    hd_out[...] = jnp.dot(h, wm2[...], preferred_element_type=F32)


def _prep_call(nf, wne, bne, wm1, wm2):
    out = [jax.ShapeDtypeStruct((N, HID), F32)] * 3
    return pl.pallas_call(_prep_body, out_shape=out)(nf, wne, bne, wm1, wm2)


_EM_TILE = 4096


def _em_body(ef, wee, bee, wm3c, bmc, em0, em1, em2):
    wem = jnp.dot(wee[...], wm3c[...], preferred_element_type=F32)
    bem = jnp.dot(bee[...], wm3c[...], preferred_element_type=F32) + bmc[...]
    z = jnp.dot(ef[...], wem, preferred_element_type=F32) + bem
    em0[...] = z[:, :HID]
    em1[...] = z[:, HID:2 * HID]
    em2[...] = z[:, 2 * HID:]


def _em_call(ef, wee, bee, wm3c, bmc):
    grid = (E // _EM_TILE,)
    out = [jax.ShapeDtypeStruct((E, HID), F32)] * 3
    return pl.pallas_call(
        _em_body,
        grid=grid,
        in_specs=[
            pl.BlockSpec((_EM_TILE, 16), lambda i: (i, 0)),
            pl.BlockSpec((16, HID), lambda i: (0, 0)),
            pl.BlockSpec((1, HID), lambda i: (0, 0)),
            pl.BlockSpec((HID, 3 * HID), lambda i: (0, 0)),
            pl.BlockSpec((1, 3 * HID), lambda i: (0, 0)),
        ],
        out_specs=[pl.BlockSpec((_EM_TILE, HID), lambda i: (i, 0))] * 3,
        out_shape=out,
    )(ef, wee, bee, wm3c, bmc)


def _upd_first_body(h, accp, cntp, wu1, wu2, bu, wn1, wn2,
                    hn_out, hsn_out, hdn_out, rinv_out):
    cw = cntp[...]
    cnt = cw[0, :, 0:1] + cw[1, :, 0:1]
    rinv = 1.0 / (cnt + 1e-6)
    rinv_out[...] = rinv
    agg = (accp[0] + accp[1]) * rinv
    hn = jnp.maximum(
        jnp.dot(h[...], wu1[...], preferred_element_type=F32)
        + jnp.dot(agg, wu2[...], preferred_element_type=F32) + bu[...], 0.0)
    hn_out[...] = hn
    hsn_out[...] = jnp.dot(hn, wn1[...], preferred_element_type=F32)
    hdn_out[...] = jnp.dot(hn, wn2[...], preferred_element_type=F32)


def _upd_mid_body(h, accp, rinv, wu1, wu2, bu, wn1, wn2,
                  hn_out, hsn_out, hdn_out):
    agg = (accp[0] + accp[1]) * rinv[...]
    hn = jnp.maximum(
        jnp.dot(h[...], wu1[...], preferred_element_type=F32)
        + jnp.dot(agg, wu2[...], preferred_element_type=F32) + bu[...], 0.0)
    hn_out[...] = hn
    hsn_out[...] = jnp.dot(hn, wn1[...], preferred_element_type=F32)
    hdn_out[...] = jnp.dot(hn, wn2[...], preferred_element_type=F32)


def _upd_last_body(h, accp, rinv, wu1, wu2, bu, wqkv, bqkv, qkv_out):
    agg = (accp[0] + accp[1]) * rinv[...]
    hn = jnp.maximum(
        jnp.dot(h[...], wu1[...], preferred_element_type=F32)
        + jnp.dot(agg, wu2[...], preferred_element_type=F32) + bu[...], 0.0)
    qkv_out[...] = jnp.dot(hn, wqkv[...], preferred_element_type=F32) + bqkv[...]


_TQ = 256


def _attn_body(qkv_q, qkv_all, wo, bo, wh4, bh4, att_out, attw_out, heads_out):
    q = qkv_q[...]
    kv = qkv_all[...]
    accw = jnp.zeros((_TQ, N), F32)
    ctxs = []
    for hh in range(NH):
        qh = q[:, hh * HD:(hh + 1) * HD]
        kh = kv[:, HID + hh * HD: HID + (hh + 1) * HD]
        vh = kv[:, 2 * HID + hh * HD: 2 * HID + (hh + 1) * HD]
        s = lax.dot_general(qh, kh, (((1,), (1,)), ((), ())),
                            preferred_element_type=F32) * 0.25
        s = s - jnp.max(s, axis=1, keepdims=True)
        pexp = jnp.exp(s)
        attn = pexp / jnp.sum(pexp, axis=1, keepdims=True)
        accw = accw + attn
        ctxs.append(jnp.dot(attn, vh, preferred_element_type=F32))
    ctx = jnp.concatenate(ctxs, axis=1)
    att = jnp.dot(ctx, wo[...], preferred_element_type=F32) + bo[...]
    att_out[...] = att
    attw_out[...] = accw * (1.0 / NH)
    z = jnp.dot(att, wh4[...], preferred_element_type=F32) + bh4[...]
    sig = 1.0 / (1.0 + jnp.exp(-z))
    heads_out[...] = jnp.concatenate([sig[:, :2], z[:, 2:]], axis=1)


def _attn_call(qkv, wo, bo, wh4, bh4):
    grid = (N // _TQ,)
    out = [
        jax.ShapeDtypeStruct((N, HID), F32),
        jax.ShapeDtypeStruct((N, N), F32),
        jax.ShapeDtypeStruct((N, 4), F32),
    ]
    return pl.pallas_call(
        _attn_body,
        grid=grid,
        in_specs=[
            pl.BlockSpec((_TQ, 3 * HID), lambda i: (i, 0)),
            pl.BlockSpec((N, 3 * HID), lambda i: (0, 0)),
            pl.BlockSpec((HID, HID), lambda i: (0, 0)),
            pl.BlockSpec((1, HID), lambda i: (0, 0)),
            pl.BlockSpec((HID, 4), lambda i: (0, 0)),
            pl.BlockSpec((1, 4), lambda i: (0, 0)),
        ],
        out_specs=[
            pl.BlockSpec((_TQ, HID), lambda i: (i, 0)),
            pl.BlockSpec((_TQ, N), lambda i: (i, 0)),
            pl.BlockSpec((_TQ, 4), lambda i: (i, 0)),
        ],
        out_shape=out,
    )(qkv, qkv, wo, bo, wh4, bh4)


def _make_sc_layer():
    out_type = jax.ShapeDtypeStruct((NC, N, HID), F32)
    scratch = [
        pltpu.VMEM((CHUNK,), jnp.int32),
        pltpu.VMEM((CHUNK,), jnp.int32),
        pltpu.VMEM((CHUNK, HID), F32),
        pltpu.VMEM((CHUNK, HID), F32),
        pltpu.VMEM((CHUNK, HID), F32),
        pltpu.VMEM_SHARED((N, HID), F32),
        pltpu.SemaphoreType.DMA,
        pltpu.SemaphoreType.DMA,
        pltpu.SemaphoreType.DMA,
    ]

    def body(row_h, col_h, hs_h, hd_h, em_h, acc_out,
             riv, civ, srcv, dstv, emv, accS, s1, s2, s3):
        cid = lax.axis_index("c")
        sid = lax.axis_index("s")
        wid = cid * NS + sid
        base_r = sid * RPT

        zero = jnp.zeros((LANES,), F32)

        def zb(i, _):
            e = i // VPR
            j = lax.rem(i, VPR) * LANES
            srcv[e, pl.ds(j, LANES)] = zero
            return 0
        lax.fori_loop(0, CHUNK * VPR, zb, 0)
        pltpu.sync_copy(srcv, accS.at[pl.ds(base_r, CHUNK)])
        pltpu.sync_copy(srcv, accS.at[pl.ds(base_r + CHUNK, CHUNK)])
        plsc.subcore_barrier()

        def chunk_body(t, _):
            base = wid * EPW + t * CHUNK
            pltpu.sync_copy(row_h.at[pl.ds(base, CHUNK)], riv)
            pltpu.sync_copy(col_h.at[pl.ds(base, CHUNK)], civ)
            c1 = pltpu.async_copy(hs_h.at[riv], srcv, s1)
            c2 = pltpu.async_copy(hd_h.at[civ], dstv, s2)
            c3 = pltpu.async_copy(em_h.at[pl.ds(base, CHUNK)], emv, s3)
            c1.wait()
            c2.wait()
            c3.wait()

            def eb(e, _):
                for j in range(VPR):
                    sl = pl.ds(j * LANES, LANES)
                    x = srcv[e, sl] + dstv[e, sl] + emv[e, sl]
                    emv[e, sl] = jnp.maximum(x, 0.0)
                return 0
            lax.fori_loop(0, CHUNK, eb, 0)
            pltpu.sync_copy(emv, accS.at[civ], add=True)
            return 0
        lax.fori_loop(0, NCHUNK, chunk_body, 0)
        plsc.subcore_barrier()
        pltpu.sync_copy(accS.at[pl.ds(base_r, RPT)],
                        acc_out.at[cid, pl.ds(base_r, RPT)])

    return pl.kernel(body, out_type=out_type,
                     mesh=_sc_mesh(), scratch_types=scratch)


def _make_sc_cnt():
    out_type = jax.ShapeDtypeStruct((NC, N, HID), F32)
    scratch = [
        pltpu.VMEM((CHUNK,), jnp.int32),
        pltpu.VMEM((CHUNK, HID), F32),
        pltpu.VMEM((CHUNK, HID), F32),
        pltpu.VMEM_SHARED((N, HID), F32),
    ]

    def body(col_h, cnt_out, civ, zv, onesv, cntS):
        cid = lax.axis_index("c")
        sid = lax.axis_index("s")
        wid = cid * NS + sid
        base_r = sid * RPT

        zero = jnp.zeros((LANES,), F32)
        one = jnp.ones((LANES,), F32)

        def zb(i, _):
            e = i // VPR
            j = lax.rem(i, VPR) * LANES
            zv[e, pl.ds(j, LANES)] = zero
            onesv[e, pl.ds(j, LANES)] = one
            return 0
        lax.fori_loop(0, CHUNK * VPR, zb, 0)
        pltpu.sync_copy(zv, cntS.at[pl.ds(base_r, CHUNK)])
        pltpu.sync_copy(zv, cntS.at[pl.ds(base_r + CHUNK, CHUNK)])
        plsc.subcore_barrier()

        def chunk_body(t, _):
            base = wid * EPW + t * CHUNK
            pltpu.sync_copy(col_h.at[pl.ds(base, CHUNK)], civ)
            pltpu.sync_copy(onesv, cntS.at[civ], add=True)
            return 0
        lax.fori_loop(0, NCHUNK, chunk_body, 0)
        plsc.subcore_barrier()
        pltpu.sync_copy(cntS.at[pl.ds(base_r, RPT)],
                        cnt_out.at[cid, pl.ds(base_r, RPT)])

    return pl.kernel(body, out_type=out_type,
                     mesh=_sc_mesh(), scratch_types=scratch)


def _make_sc_atk():
    scratch = [
        pltpu.VMEM((N,), F32),
        pltpu.VMEM((N,), F32),
        pltpu.VMEM((EPW,), jnp.int32),
        pltpu.VMEM((EPW,), jnp.int32),
        pltpu.VMEM((EPW,), F32),
    ]

    def body(asrc_h, adst_h, row_h, col_h, out_h, av, bv, riv, civ, ov):
        cid = lax.axis_index("c")
        sid = lax.axis_index("s")
        wid = cid * NS + sid
        base = wid * EPW
        pltpu.sync_copy(asrc_h, av)
        pltpu.sync_copy(adst_h, bv)
        pltpu.sync_copy(row_h.at[pl.ds(base, EPW)], riv)
        pltpu.sync_copy(col_h.at[pl.ds(base, EPW)], civ)

        def eb(i, _):
            sl = pl.ds(i * LANES, LANES)
            a = plsc.load_gather(av, [riv[sl]])
            b = plsc.load_gather(bv, [civ[sl]])
            x = a + b
            ov[sl] = 1.0 / (1.0 + jnp.exp(-x))
            return 0
        lax.fori_loop(0, EPW // LANES, eb, 0)
        pltpu.sync_copy(ov, out_h.at[pl.ds(base, EPW)])

    return pl.kernel(body, out_type=jax.ShapeDtypeStruct((E,), F32),
                     mesh=_sc_mesh(), scratch_types=scratch,
                     compiler_params=pltpu.CompilerParams(
                         use_tc_tiling_on_sc=False,
                         needs_layout_passes=False))


def kernel(node_features, edge_features, edge_index, params):
    p = params
    layers = p['layers']
    row = edge_index[0]
    col = edge_index[1]

    wm1 = [lp['Wm'][:HID] for lp in layers]
    wm2 = [lp['Wm'][HID:2 * HID] for lp in layers]
    wm3c = jnp.concatenate([lp['Wm'][2 * HID:] for lp in layers], axis=1)
    bmc = jnp.concatenate([lp['bm'] for lp in layers])[None, :]
    wu1 = [lp['Wu'][:HID] for lp in layers]
    wu2 = [lp['Wu'][HID:] for lp in layers]
    bu = [lp['bu'][None, :] for lp in layers]
    wqkv = jnp.concatenate([p['Wq'], p['Wk'], p['Wv']], axis=1)
    bqkv = jnp.concatenate([p['bq'], p['bk'], p['bv']])[None, :]
    wh4 = jnp.concatenate(
        [p['Wvul'], p['Wdef'], p['Watk'][:HID], p['Watk'][HID:]], axis=1)
    bh4 = jnp.concatenate(
        [p['bvul'], p['bdef'], p['batk'], jnp.zeros((1,), F32)])[None, :]

    h, hs, hd = _prep_call(node_features, p['W_ne'], p['b_ne'][None, :],
                           wm1[0], wm2[0])
    ems = _em_call(edge_features, p['W_ee'], p['b_ee'][None, :], wm3c, bmc)

    sc_layer = _make_sc_layer()
    sc_cnt = _make_sc_cnt()

    upd_first = pl.pallas_call(_upd_first_body, out_shape=[
        jax.ShapeDtypeStruct((N, HID), F32),
        jax.ShapeDtypeStruct((N, HID), F32),
        jax.ShapeDtypeStruct((N, HID), F32),
        jax.ShapeDtypeStruct((N, 1), F32),
    ])
    upd_mid = pl.pallas_call(_upd_mid_body, out_shape=[
        jax.ShapeDtypeStruct((N, HID), F32),
        jax.ShapeDtypeStruct((N, HID), F32),
        jax.ShapeDtypeStruct((N, HID), F32),
    ])
    upd_last = pl.pallas_call(_upd_last_body, out_shape=[
        jax.ShapeDtypeStruct((N, 3 * HID), F32),
    ])

    cntp = sc_cnt(col)
    accp = sc_layer(row, col, hs, hd, ems[0])
    h, hs, hd, rinv = upd_first(h, accp, cntp, wu1[0], wu2[0], bu[0],
                                wm1[1], wm2[1])

    accp = sc_layer(row, col, hs, hd, ems[1])
    h, hs, hd = upd_mid(h, accp, rinv, wu1[1], wu2[1], bu[1],
                        wm1[2], wm2[2])

    accp = sc_layer(row, col, hs, hd, ems[2])
    qkv = upd_last(h, accp, rinv, wu1[2], wu2[2], bu[2], wqkv, bqkv)[0]

    attended, attn_w, heads = _attn_call(qkv, p['Wo'], p['bo'][None, :],
                                         wh4, bh4)
    vuln = heads[:, 0:1]
    defp = heads[:, 1:2]
    asrc = heads[:, 2]
    adst = heads[:, 3]

    atk = _make_sc_atk()(asrc, adst, row, col)[:, None]
    return (attended, vuln, defp, atk, attn_w)


# speedup vs baseline: 5.7238x; 1.0871x over previous
import functools

import jax
import jax.numpy as jnp
from jax import lax
from jax.experimental import pallas as pl
from jax.experimental.pallas import tpu as pltpu
from jax.experimental.pallas import tpu_sc as plsc

N = 4096
E = 131072
HID = 128
NH = 8
HD = 16

NC = 2
NS = 16
LANES = 16
NW = NC * NS
EPW = E // NW
CHUNK = 128
NCHUNK = EPW // CHUNK
RPT = N // NS
VPR = HID // LANES

F32 = jnp.float32


def _sc_mesh():
    return plsc.VectorSubcoreMesh(core_axis_name="c", subcore_axis_name="s")


def _prep_body(nf, wne, bne, wm1, wm2, h_out, hs_out, hd_out):
    h = jnp.dot(nf[...], wne[...], preferred_element_type=F32) + bne[...]
    h_out[...] = h
    hs_out[...] = jnp.dot(h, wm1[...], preferred_element_type=F32)
    hd_out[...] = jnp.dot(h, wm2[...], preferred_element_type=F32)


def _prep_call(nf, wne, bne, wm1, wm2):
    out = [jax.ShapeDtypeStruct((N, HID), F32)] * 3
    return pl.pallas_call(_prep_body, out_shape=out)(nf, wne, bne, wm1, wm2)


_EM_TILE = 4096


def _em_body(ef, wee, bee, wm3c, bmc, em0, em1, em2):
    wem = jnp.dot(wee[...], wm3c[...], preferred_element_type=F32)
    bem = jnp.dot(bee[...], wm3c[...], preferred_element_type=F32) + bmc[...]
    z = jnp.dot(ef[...], wem, preferred_element_type=F32) + bem
    em0[...] = z[:, :HID]
    em1[...] = z[:, HID:2 * HID]
    em2[...] = z[:, 2 * HID:]


def _em_call(ef, wee, bee, wm3c, bmc):
    grid = (E // _EM_TILE,)
    out = [jax.ShapeDtypeStruct((E, HID), F32)] * 3
    return pl.pallas_call(
        _em_body,
        grid=grid,
        in_specs=[
            pl.BlockSpec((_EM_TILE, 16), lambda i: (i, 0)),
            pl.BlockSpec((16, HID), lambda i: (0, 0)),
            pl.BlockSpec((1, HID), lambda i: (0, 0)),
            pl.BlockSpec((HID, 3 * HID), lambda i: (0, 0)),
            pl.BlockSpec((1, 3 * HID), lambda i: (0, 0)),
        ],
        out_specs=[pl.BlockSpec((_EM_TILE, HID), lambda i: (i, 0))] * 3,
        out_shape=out,
    )(ef, wee, bee, wm3c, bmc)


def _upd_first_body(h, accp, cntp, wu1, wu2, bu, wn1, wn2,
                    hn_out, hsn_out, hdn_out, rinv_out):
    cw = cntp[...]
    cnt = cw[0, :, 0:1] + cw[1, :, 0:1]
    rinv = 1.0 / (cnt + 1e-6)
    rinv_out[...] = rinv
    agg = (accp[0] + accp[1]) * rinv
    hn = jnp.maximum(
        jnp.dot(h[...], wu1[...], preferred_element_type=F32)
        + jnp.dot(agg, wu2[...], preferred_element_type=F32) + bu[...], 0.0)
    hn_out[...] = hn
    hsn_out[...] = jnp.dot(hn, wn1[...], preferred_element_type=F32)
    hdn_out[...] = jnp.dot(hn, wn2[...], preferred_element_type=F32)


def _upd_mid_body(h, accp, rinv, wu1, wu2, bu, wn1, wn2,
                  hn_out, hsn_out, hdn_out):
    agg = (accp[0] + accp[1]) * rinv[...]
    hn = jnp.maximum(
        jnp.dot(h[...], wu1[...], preferred_element_type=F32)
        + jnp.dot(agg, wu2[...], preferred_element_type=F32) + bu[...], 0.0)
    hn_out[...] = hn
    hsn_out[...] = jnp.dot(hn, wn1[...], preferred_element_type=F32)
    hdn_out[...] = jnp.dot(hn, wn2[...], preferred_element_type=F32)


def _upd_last_body(h, accp, rinv, wu1, wu2, bu, wqkv, bqkv, qkv_out):
    agg = (accp[0] + accp[1]) * rinv[...]
    hn = jnp.maximum(
        jnp.dot(h[...], wu1[...], preferred_element_type=F32)
        + jnp.dot(agg, wu2[...], preferred_element_type=F32) + bu[...], 0.0)
    qkv_out[...] = jnp.dot(hn, wqkv[...], preferred_element_type=F32) + bqkv[...]


_TQ = 256


def _attn_body(qkv_q, qkv_all, wo, bo, wh4, bh4, att_out, attw_out, heads_out):
    q = qkv_q[...]
    kv = qkv_all[...]
    accw = jnp.zeros((_TQ, N), F32)
    ctxs = []
    for hh in range(NH):
        qh = q[:, hh * HD:(hh + 1) * HD]
        kh = kv[:, HID + hh * HD: HID + (hh + 1) * HD]
        vh = kv[:, 2 * HID + hh * HD: 2 * HID + (hh + 1) * HD]
        s = lax.dot_general(qh, kh, (((1,), (1,)), ((), ())),
                            preferred_element_type=F32) * 0.25
        s = s - jnp.max(s, axis=1, keepdims=True)
        pexp = jnp.exp(s)
        attn = pexp / jnp.sum(pexp, axis=1, keepdims=True)
        accw = accw + attn
        ctxs.append(jnp.dot(attn, vh, preferred_element_type=F32))
    ctx = jnp.concatenate(ctxs, axis=1)
    att = jnp.dot(ctx, wo[...], preferred_element_type=F32) + bo[...]
    att_out[...] = att
    attw_out[...] = accw * (1.0 / NH)
    z = jnp.dot(att, wh4[...], preferred_element_type=F32) + bh4[...]
    sig = 1.0 / (1.0 + jnp.exp(-z))
    heads_out[...] = jnp.concatenate([sig[:, :2], z[:, 2:]], axis=1)


def _attn_call(qkv, wo, bo, wh4, bh4):
    grid = (N // _TQ,)
    out = [
        jax.ShapeDtypeStruct((N, HID), F32),
        jax.ShapeDtypeStruct((N, N), F32),
        jax.ShapeDtypeStruct((N, 4), F32),
    ]
    return pl.pallas_call(
        _attn_body,
        grid=grid,
        in_specs=[
            pl.BlockSpec((_TQ, 3 * HID), lambda i: (i, 0)),
            pl.BlockSpec((N, 3 * HID), lambda i: (0, 0)),
            pl.BlockSpec((HID, HID), lambda i: (0, 0)),
            pl.BlockSpec((1, HID), lambda i: (0, 0)),
            pl.BlockSpec((HID, 4), lambda i: (0, 0)),
            pl.BlockSpec((1, 4), lambda i: (0, 0)),
        ],
        out_specs=[
            pl.BlockSpec((_TQ, HID), lambda i: (i, 0)),
            pl.BlockSpec((_TQ, N), lambda i: (i, 0)),
            pl.BlockSpec((_TQ, 4), lambda i: (i, 0)),
        ],
        out_shape=out,
    )(qkv, qkv, wo, bo, wh4, bh4)


_LCHUNK = 64
_LNCHUNK = EPW // _LCHUNK


def _make_sc_layer():
    out_type = jax.ShapeDtypeStruct((NC, N, HID), F32)
    scratch = (
        [pltpu.VMEM((_LCHUNK,), jnp.int32)] * 4 +
        [pltpu.VMEM((_LCHUNK, HID), F32)] * 6 +
        [pltpu.VMEM_SHARED((N, HID), F32)] +
        [pltpu.SemaphoreType.DMA] * 6
    )

    def body(row_h, col_h, hs_h, hd_h, em_h, acc_out,
             riv0, riv1, civ0, civ1, srcv0, srcv1, dstv0, dstv1,
             emv0, emv1, accS, s10, s11, s20, s21, s30, s31):
        riv = [riv0, riv1]
        civ = [civ0, civ1]
        srcv = [srcv0, srcv1]
        dstv = [dstv0, dstv1]
        emv = [emv0, emv1]
        s1 = [s10, s11]
        s2 = [s20, s21]
        s3 = [s30, s31]
        cid = lax.axis_index("c")
        sid = lax.axis_index("s")
        wid = cid * NS + sid
        base_r = sid * RPT

        zero = jnp.zeros((LANES,), F32)

        def zb(i, _):
            e = i // VPR
            j = lax.rem(i, VPR) * LANES
            srcv0[e, pl.ds(j, LANES)] = zero
            return 0
        lax.fori_loop(0, _LCHUNK * VPR, zb, 0)
        for q in range(RPT // _LCHUNK):
            pltpu.sync_copy(srcv0, accS.at[pl.ds(base_r + q * _LCHUNK, _LCHUNK)])
        plsc.subcore_barrier()

        def issue(c, b):
            base = wid * EPW + c * _LCHUNK
            pltpu.sync_copy(row_h.at[pl.ds(base, _LCHUNK)], riv[b])
            pltpu.sync_copy(col_h.at[pl.ds(base, _LCHUNK)], civ[b])
            pltpu.async_copy(hs_h.at[riv[b]], srcv[b], s1[b])
            pltpu.async_copy(hd_h.at[civ[b]], dstv[b], s2[b])
            pltpu.async_copy(em_h.at[pl.ds(base, _LCHUNK)], emv[b], s3[b])

        def wait(b):
            pltpu.make_async_copy(hs_h.at[riv[b]], srcv[b], s1[b]).wait()
            pltpu.make_async_copy(hd_h.at[civ[b]], dstv[b], s2[b]).wait()
            pltpu.make_async_copy(em_h.at[pl.ds(0, _LCHUNK)], emv[b], s3[b]).wait()

        issue(0, 0)

        def outer(u, _):
            for b in range(2):
                c = 2 * u + b
                wait(b)
                issue(lax.rem(c + 1, _LNCHUNK), 1 - b)

                def eb(e, _):
                    for j in range(VPR):
                        sl = pl.ds(j * LANES, LANES)
                        x = srcv[b][e, sl] + dstv[b][e, sl] + emv[b][e, sl]
                        emv[b][e, sl] = jnp.maximum(x, 0.0)
                    return 0
                lax.fori_loop(0, _LCHUNK, eb, 0)
                pltpu.sync_copy(emv[b], accS.at[civ[b]], add=True)
            return 0
        lax.fori_loop(0, _LNCHUNK // 2, outer, 0)
        wait(0)
        plsc.subcore_barrier()
        pltpu.sync_copy(accS.at[pl.ds(base_r, RPT)],
                        acc_out.at[cid, pl.ds(base_r, RPT)])

    return pl.kernel(body, out_type=out_type,
                     mesh=_sc_mesh(), scratch_types=scratch)


def _make_sc_cnt():
    out_type = jax.ShapeDtypeStruct((NC, N, HID), F32)
    scratch = [
        pltpu.VMEM((CHUNK,), jnp.int32),
        pltpu.VMEM((CHUNK, HID), F32),
        pltpu.VMEM((CHUNK, HID), F32),
        pltpu.VMEM_SHARED((N, HID), F32),
    ]

    def body(col_h, cnt_out, civ, zv, onesv, cntS):
        cid = lax.axis_index("c")
        sid = lax.axis_index("s")
        wid = cid * NS + sid
        base_r = sid * RPT

        zero = jnp.zeros((LANES,), F32)
        one = jnp.ones((LANES,), F32)

        def zb(i, _):
            e = i // VPR
            j = lax.rem(i, VPR) * LANES
            zv[e, pl.ds(j, LANES)] = zero
            onesv[e, pl.ds(j, LANES)] = one
            return 0
        lax.fori_loop(0, CHUNK * VPR, zb, 0)
        pltpu.sync_copy(zv, cntS.at[pl.ds(base_r, CHUNK)])
        pltpu.sync_copy(zv, cntS.at[pl.ds(base_r + CHUNK, CHUNK)])
        plsc.subcore_barrier()

        def chunk_body(t, _):
            base = wid * EPW + t * CHUNK
            pltpu.sync_copy(col_h.at[pl.ds(base, CHUNK)], civ)
            pltpu.sync_copy(onesv, cntS.at[civ], add=True)
            return 0
        lax.fori_loop(0, NCHUNK, chunk_body, 0)
        plsc.subcore_barrier()
        pltpu.sync_copy(cntS.at[pl.ds(base_r, RPT)],
                        cnt_out.at[cid, pl.ds(base_r, RPT)])

    return pl.kernel(body, out_type=out_type,
                     mesh=_sc_mesh(), scratch_types=scratch)


def _make_sc_atk():
    scratch = [
        pltpu.VMEM((N,), F32),
        pltpu.VMEM((N,), F32),
        pltpu.VMEM((EPW,), jnp.int32),
        pltpu.VMEM((EPW,), jnp.int32),
        pltpu.VMEM((EPW,), F32),
    ]

    def body(asrc_h, adst_h, row_h, col_h, out_h, av, bv, riv, civ, ov):
        cid = lax.axis_index("c")
        sid = lax.axis_index("s")
        wid = cid * NS + sid
        base = wid * EPW
        pltpu.sync_copy(asrc_h, av)
        pltpu.sync_copy(adst_h, bv)
        pltpu.sync_copy(row_h.at[pl.ds(base, EPW)], riv)
        pltpu.sync_copy(col_h.at[pl.ds(base, EPW)], civ)

        def eb(i, _):
            sl = pl.ds(i * LANES, LANES)
            a = plsc.load_gather(av, [riv[sl]])
            b = plsc.load_gather(bv, [civ[sl]])
            x = a + b
            ov[sl] = 1.0 / (1.0 + jnp.exp(-x))
            return 0
        lax.fori_loop(0, EPW // LANES, eb, 0)
        pltpu.sync_copy(ov, out_h.at[pl.ds(base, EPW)])

    return pl.kernel(body, out_type=jax.ShapeDtypeStruct((E,), F32),
                     mesh=_sc_mesh(), scratch_types=scratch,
                     compiler_params=pltpu.CompilerParams(
                         use_tc_tiling_on_sc=False,
                         needs_layout_passes=False))


def kernel(node_features, edge_features, edge_index, params):
    p = params
    layers = p['layers']
    row = edge_index[0]
    col = edge_index[1]

    wm1 = [lp['Wm'][:HID] for lp in layers]
    wm2 = [lp['Wm'][HID:2 * HID] for lp in layers]
    wm3c = jnp.concatenate([lp['Wm'][2 * HID:] for lp in layers], axis=1)
    bmc = jnp.concatenate([lp['bm'] for lp in layers])[None, :]
    wu1 = [lp['Wu'][:HID] for lp in layers]
    wu2 = [lp['Wu'][HID:] for lp in layers]
    bu = [lp['bu'][None, :] for lp in layers]
    wqkv = jnp.concatenate([p['Wq'], p['Wk'], p['Wv']], axis=1)
    bqkv = jnp.concatenate([p['bq'], p['bk'], p['bv']])[None, :]
    wh4 = jnp.concatenate(
        [p['Wvul'], p['Wdef'], p['Watk'][:HID], p['Watk'][HID:]], axis=1)
    bh4 = jnp.concatenate(
        [p['bvul'], p['bdef'], p['batk'], jnp.zeros((1,), F32)])[None, :]

    h, hs, hd = _prep_call(node_features, p['W_ne'], p['b_ne'][None, :],
                           wm1[0], wm2[0])
    ems = _em_call(edge_features, p['W_ee'], p['b_ee'][None, :], wm3c, bmc)

    sc_layer = _make_sc_layer()
    sc_cnt = _make_sc_cnt()

    upd_first = pl.pallas_call(_upd_first_body, out_shape=[
        jax.ShapeDtypeStruct((N, HID), F32),
        jax.ShapeDtypeStruct((N, HID), F32),
        jax.ShapeDtypeStruct((N, HID), F32),
        jax.ShapeDtypeStruct((N, 1), F32),
    ])
    upd_mid = pl.pallas_call(_upd_mid_body, out_shape=[
        jax.ShapeDtypeStruct((N, HID), F32),
        jax.ShapeDtypeStruct((N, HID), F32),
        jax.ShapeDtypeStruct((N, HID), F32),
    ])
    upd_last = pl.pallas_call(_upd_last_body, out_shape=[
        jax.ShapeDtypeStruct((N, 3 * HID), F32),
    ])

    cntp = sc_cnt(col)
    accp = sc_layer(row, col, hs, hd, ems[0])
    h, hs, hd, rinv = upd_first(h, accp, cntp, wu1[0], wu2[0], bu[0],
                                wm1[1], wm2[1])

    accp = sc_layer(row, col, hs, hd, ems[1])
    h, hs, hd = upd_mid(h, accp, rinv, wu1[1], wu2[1], bu[1],
                        wm1[2], wm2[2])

    accp = sc_layer(row, col, hs, hd, ems[2])
    qkv = upd_last(h, accp, rinv, wu1[2], wu2[2], bu[2], wqkv, bqkv)[0]

    attended, attn_w, heads = _attn_call(qkv, p['Wo'], p['bo'][None, :],
                                         wh4, bh4)
    vuln = heads[:, 0:1]
    defp = heads[:, 1:2]
    asrc = heads[:, 2]
    adst = heads[:, 3]

    atk = _make_sc_atk()(asrc, adst, row, col)[:, None]
    return (attended, vuln, defp, atk, attn_w)
